# baseline (speedup 1.0000x reference)
import functools

import jax
import jax.numpy as jnp
from jax import lax
from jax.experimental import pallas as pl
from jax.experimental.pallas import tpu as pltpu
from jax.experimental.pallas import tpu_sc as plsc

NC, NS = 2, 16
NW = NC * NS
NBUF = 4


def sc_add(x, emb_table):
    B, T, D = x.shape
    per_w = B // NW
    groups = per_w // NBUF
    mesh = plsc.VectorSubcoreMesh(core_axis_name="c", subcore_axis_name="s")

    @functools.partial(
        pl.kernel,
        mesh=mesh,
        out_type=jax.ShapeDtypeStruct((B, T, D), jnp.float32),
        scratch_types=(
            [pltpu.VMEM((T, D), jnp.float32)] * (1 + NBUF)
            + [pltpu.SemaphoreType.DMA] * (2 * NBUF)
        ),
    )
    def k(x_hbm, emb_hbm, out_hbm, emb_v, *rest):
        bufs = rest[:NBUF]
        sis = rest[NBUF : 2 * NBUF]
        sos = rest[2 * NBUF :]
        wid = lax.axis_index("s") * NC + lax.axis_index("c")
        base = wid * per_w
        pltpu.sync_copy(emb_hbm, emb_v)

        def add_table(buf):
            @plsc.parallel_loop(0, T, unroll=2)
            def _(r):
                for j in range(D // 16):
                    sl = pl.ds(j * 16, 16)
                    buf[r, sl] = buf[r, sl] + emb_v[r, sl]

        for j in range(NBUF):
            pltpu.async_copy(x_hbm.at[base + j], bufs[j], sis[j])

        def body(i, carry):
            b0 = base + NBUF * i
            for j in range(NBUF):
                b = b0 + j
                pltpu.make_async_copy(x_hbm.at[b], bufs[j], sis[j]).wait()
                add_table(bufs[j])
                pltpu.async_copy(bufs[j], out_hbm.at[b], sos[j])

            @pl.when(i < groups - 1)
            def _():
                for j in range(NBUF):
                    b = b0 + j
                    pltpu.make_async_copy(bufs[j], out_hbm.at[b], sos[j]).wait()
                    pltpu.async_copy(x_hbm.at[b + NBUF], bufs[j], sis[j])

            return carry

        lax.fori_loop(0, groups, body, 0)
        last0 = base + per_w - NBUF
        for j in range(NBUF):
            pltpu.make_async_copy(bufs[j], out_hbm.at[last0 + j], sos[j]).wait()

    return k(x, emb_table[:T])


def kernel(x, emb_table):
    return sc_add(x, emb_table)


# speedup vs baseline: 1.4747x; 1.4747x over previous
import functools

import jax
import jax.numpy as jnp
from jax import lax
from jax.experimental import pallas as pl
from jax.experimental.pallas import tpu as pltpu
from jax.experimental.pallas import tpu_sc as plsc

_NC, _NS = 2, 16
_NW = _NC * _NS


def _sc_gather(emb_table, T):
    D = emb_table.shape[1]
    rows = 16
    n_w = (T + rows - 1) // rows
    mesh = plsc.VectorSubcoreMesh(
        core_axis_name="c", subcore_axis_name="s", num_cores=1
    )

    @functools.partial(
        pl.kernel,
        mesh=mesh,
        out_type=jax.ShapeDtypeStruct((T, D), jnp.float32),
        scratch_types=[
            pltpu.VMEM((rows, D), jnp.float32),
            pltpu.SemaphoreType.DMA,
        ],
    )
    def k(emb_hbm, out_hbm, rows_v, sem):
        wid = lax.axis_index("s")

        @pl.when(wid < n_w)
        def _():
            base = jnp.minimum(wid * rows, T - rows)
            idx = lax.iota(jnp.int32, rows) + base
            pltpu.async_copy(emb_hbm.at[idx], rows_v, sem).wait()
            pltpu.sync_copy(rows_v, out_hbm.at[pl.ds(base, rows)])

    return k(emb_table)


def _add_body(x_ref, emb_ref, o_ref):
    o_ref[...] = x_ref[...] + emb_ref[...][None, :, :]


def _tc_add(x, pos_emb):
    B, T, D = x.shape
    B_BLK = 128
    return pl.pallas_call(
        _add_body,
        grid=(B // B_BLK,),
        in_specs=[
            pl.BlockSpec((B_BLK, T, D), lambda i: (i, 0, 0)),
            pl.BlockSpec((T, D), lambda i: (0, 0)),
        ],
        out_specs=pl.BlockSpec((B_BLK, T, D), lambda i: (i, 0, 0)),
        out_shape=jax.ShapeDtypeStruct((B, T, D), x.dtype),
        compiler_params=pltpu.CompilerParams(
            dimension_semantics=("parallel",)
        ),
    )(x, pos_emb)


def kernel(x, emb_table):
    T = x.shape[1]
    pos_emb = _sc_gather(emb_table, T)
    return _tc_add(x, pos_emb)
